# baseline (device time: 68089 ns/iter reference)
import jax
import jax.numpy as jnp
from jax import lax
from jax.experimental import pallas as pl
from jax.experimental.pallas import tpu as pltpu

B = 2
S = 1024
S_HALF = 512
K = 1024
N = 2048


def kernel(O, Wo):
    O2 = O.reshape(B, S, K)

    def body(o_ref, w_ref, out_ref, send_buf, recv_buf, send_sem, recv_sem):
        my_x = lax.axis_index("x")
        my_y = lax.axis_index("y")
        my_z = lax.axis_index("z")
        peer = (1 - my_x, my_y, my_z)

        barrier_sem = pltpu.get_barrier_semaphore()
        pl.semaphore_signal(
            barrier_sem, inc=1, device_id=peer,
            device_id_type=pl.DeviceIdType.MESH,
        )
        pl.semaphore_wait(barrier_sem, 1)

        w = w_ref[...].astype(jnp.bfloat16)
        peer_start = (1 - my_x) * S_HALF
        my_start = my_x * S_HALF

        for b in range(B):
            o_peer = o_ref[b, pl.ds(peer_start, S_HALF), :].astype(jnp.bfloat16)
            send_buf[b, :, :] = jnp.dot(
                o_peer, w, preferred_element_type=jnp.float32
            ).astype(jnp.bfloat16)

        rdma = pltpu.make_async_remote_copy(
            src_ref=send_buf,
            dst_ref=recv_buf,
            send_sem=send_sem,
            recv_sem=recv_sem,
            device_id=peer,
            device_id_type=pl.DeviceIdType.MESH,
        )
        rdma.start()

        for b in range(B):
            o_my = o_ref[b, pl.ds(my_start, S_HALF), :].astype(jnp.bfloat16)
            out_ref[b, :, :] = jnp.dot(
                o_my, w, preferred_element_type=jnp.float32
            )

        rdma.wait()
        for b in range(B):
            out_ref[b, :, :] = out_ref[b, :, :] + recv_buf[b, :, :].astype(
                jnp.float32
            )

    return pl.pallas_call(
        body,
        out_shape=jax.ShapeDtypeStruct((B, S_HALF, N), jnp.float32),
        in_specs=[
            pl.BlockSpec(memory_space=pltpu.VMEM),
            pl.BlockSpec(memory_space=pltpu.VMEM),
        ],
        out_specs=pl.BlockSpec(memory_space=pltpu.VMEM),
        scratch_shapes=[
            pltpu.VMEM((B, S_HALF, N), jnp.bfloat16),
            pltpu.VMEM((B, S_HALF, N), jnp.bfloat16),
            pltpu.SemaphoreType.DMA,
            pltpu.SemaphoreType.DMA,
        ],
        compiler_params=pltpu.CompilerParams(collective_id=0),
    )(O2, Wo)


# device time: 63386 ns/iter; 1.0742x vs baseline; 1.0742x over previous
import jax
import jax.numpy as jnp
from jax import lax
from jax.experimental import pallas as pl
from jax.experimental.pallas import tpu as pltpu

B = 2
S = 1024
S_HALF = 512
K = 1024
N = 2048


def kernel(O, Wo):
    O2 = O.reshape(B, S, K)

    CS = 128
    NCB = S_HALF // CS
    NCHUNK = B * NCB

    def body(o_ref, w_ref, out_ref, send_buf, recv_buf, send_sems, recv_sems):
        my_x = lax.axis_index("x")
        my_y = lax.axis_index("y")
        my_z = lax.axis_index("z")
        peer = (1 - my_x, my_y, my_z)

        barrier_sem = pltpu.get_barrier_semaphore()
        pl.semaphore_signal(
            barrier_sem, inc=1, device_id=peer,
            device_id_type=pl.DeviceIdType.MESH,
        )
        pl.semaphore_wait(barrier_sem, 1)

        w = w_ref[...].astype(jnp.bfloat16)
        peer_start = (1 - my_x) * S_HALF
        my_start = my_x * S_HALF

        rdmas = []
        for idx in range(NCHUNK):
            b, c = divmod(idx, NCB)
            o_chunk = o_ref[b, pl.ds(peer_start + c * CS, CS), :].astype(
                jnp.bfloat16
            )
            send_buf[idx, :, :] = jnp.dot(
                o_chunk, w, preferred_element_type=jnp.float32
            ).astype(jnp.bfloat16)
            rdma = pltpu.make_async_remote_copy(
                src_ref=send_buf.at[idx],
                dst_ref=recv_buf.at[idx],
                send_sem=send_sems.at[idx],
                recv_sem=recv_sems.at[idx],
                device_id=peer,
                device_id_type=pl.DeviceIdType.MESH,
            )
            rdma.start()
            rdmas.append(rdma)

        for b in range(B):
            o_my = o_ref[b, pl.ds(my_start, S_HALF), :].astype(jnp.bfloat16)
            out_ref[b, :, :] = jnp.dot(
                o_my, w, preferred_element_type=jnp.float32
            )

        for idx, rdma in enumerate(rdmas):
            b, c = divmod(idx, NCB)
            rdma.wait_send()
            rdma.wait_recv()
            sl = pl.ds(c * CS, CS)
            out_ref[b, sl, :] = out_ref[b, sl, :] + recv_buf[idx].astype(
                jnp.float32
            )

    return pl.pallas_call(
        body,
        out_shape=jax.ShapeDtypeStruct((B, S_HALF, N), jnp.float32),
        in_specs=[
            pl.BlockSpec(memory_space=pltpu.VMEM),
            pl.BlockSpec(memory_space=pltpu.VMEM),
        ],
        out_specs=pl.BlockSpec(memory_space=pltpu.VMEM),
        scratch_shapes=[
            pltpu.VMEM((NCHUNK, CS, N), jnp.bfloat16),
            pltpu.VMEM((NCHUNK, CS, N), jnp.bfloat16),
            pltpu.SemaphoreType.DMA((NCHUNK,)),
            pltpu.SemaphoreType.DMA((NCHUNK,)),
        ],
        compiler_params=pltpu.CompilerParams(collective_id=0),
    )(O2, Wo)
